# Initial kernel scaffold; baseline (speedup 1.0000x reference)
import jax
import jax.numpy as jnp
from jax.experimental import pallas as pl


def kernel(x, edge_index, batch, W1, a_src1, a_dst1, b1, W2, a_src2, a_dst2, b2, fcW, fcb):
    raise NotImplementedError("write your pallas kernel here")


# speedup vs baseline: 8.6007x; 8.6007x over previous
import functools

import jax
import jax.numpy as jnp
from jax import lax
from jax.experimental import pallas as pl
from jax.experimental.pallas import tpu as pltpu
from jax.experimental.pallas import tpu_sc as plsc

N = 10000
E = 160000
D = 256
HID = 256
HID2 = 512
OUT = 16
G = 64

NC, NS, L = 2, 16, 16
NW = NC * NS
EP = E + N
EP_PAD = -(-EP // (NW * L)) * (NW * L)
CHUNK = EP_PAD // NS
K = 64

W1C = HID + L
W2C = HID2 + L
BN = 1000


def _dense1_body(x_ref, w_ref, asv_ref, adv_ref, one_ref, ht_ref, as_ref, ad_ref):
    h = jnp.dot(x_ref[...], w_ref[...], preferred_element_type=jnp.float32)
    h = h + one_ref[...]
    ht_ref[...] = h
    as_ref[...] = jnp.dot(h, asv_ref[...], preferred_element_type=jnp.float32)
    ad_ref[...] = jnp.dot(h, adv_ref[...], preferred_element_type=jnp.float32)


_dense1 = pl.pallas_call(
    _dense1_body,
    grid=(N // BN,),
    in_specs=[
        pl.BlockSpec((BN, D), lambda i: (i, 0)),
        pl.BlockSpec((D, W1C), lambda i: (0, 0)),
        pl.BlockSpec((W1C, 1), lambda i: (0, 0)),
        pl.BlockSpec((W1C, 1), lambda i: (0, 0)),
        pl.BlockSpec((1, W1C), lambda i: (0, 0)),
    ],
    out_specs=[
        pl.BlockSpec((BN, W1C), lambda i: (i, 0)),
        pl.BlockSpec((BN, 1), lambda i: (i, 0)),
        pl.BlockSpec((BN, 1), lambda i: (i, 0)),
    ],
    out_shape=[
        jax.ShapeDtypeStruct((N, W1C), jnp.float32),
        jax.ShapeDtypeStruct((N, 1), jnp.float32),
        jax.ShapeDtypeStruct((N, 1), jnp.float32),
    ],
)


def _dense2_body(acc_ref, b_ref, w_ref, asv_ref, adv_ref, one_ref,
                 ht_ref, as_ref, ad_ref):
    accb = acc_ref[...]
    xin = jnp.maximum(
        accb[:, :HID] / (accb[:, HID:HID + 1] + 1e-16) + b_ref[...], 0.0)
    h = jnp.dot(xin, w_ref[...], preferred_element_type=jnp.float32)
    h = h + one_ref[...]
    ht_ref[...] = h
    as_ref[...] = jnp.dot(h, asv_ref[...], preferred_element_type=jnp.float32)
    ad_ref[...] = jnp.dot(h, adv_ref[...], preferred_element_type=jnp.float32)


_dense2 = pl.pallas_call(
    _dense2_body,
    grid=(N // BN,),
    in_specs=[
        pl.BlockSpec((BN, W1C), lambda i: (i, 0)),
        pl.BlockSpec((1, HID), lambda i: (0, 0)),
        pl.BlockSpec((HID, W2C), lambda i: (0, 0)),
        pl.BlockSpec((W2C, 1), lambda i: (0, 0)),
        pl.BlockSpec((W2C, 1), lambda i: (0, 0)),
        pl.BlockSpec((1, W2C), lambda i: (0, 0)),
    ],
    out_specs=[
        pl.BlockSpec((BN, W2C), lambda i: (i, 0)),
        pl.BlockSpec((BN, 1), lambda i: (i, 0)),
        pl.BlockSpec((BN, 1), lambda i: (i, 0)),
    ],
    out_shape=[
        jax.ShapeDtypeStruct((N, W2C), jnp.float32),
        jax.ShapeDtypeStruct((N, 1), jnp.float32),
        jax.ShapeDtypeStruct((N, 1), jnp.float32),
    ],
)


def _final_body(acc_ref, b_ref, batch_ref, fcw_ref, fcb_ref, out_ref, pool_ref):
    i = pl.program_id(0)

    @pl.when(i == 0)
    def _():
        pool_ref[...] = jnp.zeros_like(pool_ref)

    accb = acc_ref[...]
    xin = jnp.maximum(
        accb[:, :HID2] / (accb[:, HID2:HID2 + 1] + 1e-16) + b_ref[...], 0.0)
    bvec = batch_ref[...]
    oh = (bvec == lax.broadcasted_iota(jnp.int32, (BN, G), 1)).astype(jnp.float32)
    pool_ref[...] += lax.dot_general(
        oh, xin, (((0,), (0,)), ((), ())), preferred_element_type=jnp.float32)

    @pl.when(i == pl.num_programs(0) - 1)
    def _():
        logits = jnp.dot(pool_ref[...], fcw_ref[...],
                         preferred_element_type=jnp.float32) + fcb_ref[...]
        out_ref[...] = jax.nn.sigmoid(logits)


_final = pl.pallas_call(
    _final_body,
    grid=(N // BN,),
    in_specs=[
        pl.BlockSpec((BN, W2C), lambda i: (i, 0)),
        pl.BlockSpec((1, HID2), lambda i: (0, 0)),
        pl.BlockSpec((BN, 1), lambda i: (i, 0)),
        pl.BlockSpec((HID2, OUT), lambda i: (0, 0)),
        pl.BlockSpec((1, OUT), lambda i: (0, 0)),
    ],
    out_specs=pl.BlockSpec((G, OUT), lambda i: (0, 0)),
    out_shape=jax.ShapeDtypeStruct((G, OUT), jnp.float32),
    scratch_shapes=[pltpu.VMEM((G, HID2), jnp.float32)],
)


@functools.lru_cache(maxsize=None)
def _make_sc_layer(Wc, passes):
    PR = N // NC
    RMAX = max(r for _, r in passes)
    RS = -(-(RMAX + 1) // 128) * 128
    ZT = RS // NS
    NB = CHUNK // L
    mesh = plsc.VectorSubcoreMesh(core_axis_name="c", subcore_axis_name="s",
                                  num_cores=NC, num_subcores=NS)

    @functools.partial(
        pl.kernel,
        out_type=jax.ShapeDtypeStruct((N, Wc), jnp.float32),
        mesh=mesh,
        compiler_params=pltpu.CompilerParams(
            needs_layout_passes=False, use_tc_tiling_on_sc=False),
        scratch_types=[
            pltpu.VMEM((N,), jnp.float32),
            pltpu.VMEM((N,), jnp.float32),
            pltpu.VMEM((CHUNK,), jnp.int32),
            pltpu.VMEM((CHUNK,), jnp.int32),
            pltpu.VMEM((CHUNK + K + L,), jnp.int32),
            pltpu.VMEM((CHUNK + K + L,), jnp.int32),
            pltpu.VMEM((CHUNK + K + L,), jnp.float32),
            pltpu.VMEM((K, Wc), jnp.float32),
            pltpu.VMEM((K,), jnp.int32),
            pltpu.VMEM((K,), jnp.int32),
            pltpu.VMEM_SHARED((RS, Wc), jnp.float32),
        ],
    )
    def sc_layer(ht, asrc, adst, srcs, dsts, zrows, acc,
                 as_v, ad_v, src_v, dst_v, csrc_v, cdst_v, cw_v,
                 rows_v, sidx_v, didx_v, slab):
        cid = lax.axis_index("c")
        sid = lax.axis_index("s")
        base_e = pl.multiple_of(sid * CHUNK, CHUNK)
        pltpu.sync_copy(asrc, as_v)
        pltpu.sync_copy(adst, ad_v)
        pltpu.sync_copy(srcs.at[pl.ds(base_e, CHUNK)], src_v)
        pltpu.sync_copy(dsts.at[pl.ds(base_e, CHUNK)], dst_v)

        for p, (poff, R) in enumerate(passes):
            base_n = pl.multiple_of(cid * PR + poff, 8)
            pltpu.sync_copy(zrows, rows_v)
            z0 = pl.multiple_of(sid * ZT, 8)
            off = 0
            while off < ZT:
                step = min(K, ZT - off)
                pltpu.sync_copy(rows_v.at[pl.ds(0, step)],
                                slab.at[pl.ds(z0 + off, step)])
                off += step
            plsc.subcore_barrier()

            def cbody(i, cnt):
                st = pl.multiple_of(i * L, L)
                s = src_v[pl.ds(st, L)]
                d = dst_v[pl.ds(st, L)]
                m = (d >= base_n) & (d < base_n + R)
                mi = m.astype(jnp.int32)
                pos = jnp.where(m, cnt + plsc.cumsum(mi) - 1, CHUNK + K)
                plsc.store_scatter(csrc_v, [pos], s)
                plsc.store_scatter(cdst_v, [pos], d - base_n)
                return cnt + jnp.sum(mi)
            cnt = lax.fori_loop(0, NB, cbody, jnp.int32(0))

            zsrc = jnp.zeros((L,), jnp.int32)
            trash = jnp.full((L,), RS - 1, jnp.int32)
            for t in range(K // L):
                csrc_v[pl.ds(cnt + t * L, L)] = zsrc
                cdst_v[pl.ds(cnt + t * L, L)] = trash
            cntk = ((cnt + K - 1) // K) * K

            def wbody(j, carry):
                st = pl.multiple_of(j * L, L)
                s = csrc_v[pl.ds(st, L)]
                dg = jnp.minimum(cdst_v[pl.ds(st, L)] + base_n, N - 1)
                e = plsc.load_gather(as_v, [s]) + plsc.load_gather(ad_v, [dg])
                e = jnp.where(e >= 0.0, e, 0.2 * e)
                cw_v[pl.ds(st, L)] = jnp.exp(e)
                return carry
            lax.fori_loop(0, cntk // L, wbody, jnp.int32(0))

            def bbody(bi, carry):
                b0 = pl.multiple_of(bi * K, K)
                for t in range(K // L):
                    sidx_v[pl.ds(t * L, L)] = csrc_v[pl.ds(b0 + t * L, L)]
                    didx_v[pl.ds(t * L, L)] = cdst_v[pl.ds(b0 + t * L, L)]
                pltpu.sync_copy(ht.at[sidx_v], rows_v)

                def sbody(k, c2):
                    wv = plsc.load_gather(
                        cw_v, [jnp.full((L,), b0 + k, jnp.int32)])
                    for j in range(Wc // L):
                        rows_v[k, pl.ds(j * L, L)] = (
                            rows_v[k, pl.ds(j * L, L)] * wv)
                    return c2
                lax.fori_loop(0, K, sbody, jnp.int32(0))
                pltpu.sync_copy(rows_v, slab.at[didx_v], add=True)
                return carry
            lax.fori_loop(0, cntk // K, bbody, jnp.int32(0))
            plsc.subcore_barrier()

            @pl.when(sid == 0)
            def _():
                pltpu.sync_copy(slab.at[pl.ds(0, R)],
                                acc.at[pl.ds(base_n, R)])
            plsc.subcore_barrier()

    return sc_layer


def kernel(x, edge_index, batch, W1, a_src1, a_dst1, b1,
           W2, a_src2, a_dst2, b2, fcW, fcb):
    f32, i32 = jnp.float32, jnp.int32
    src = edge_index[0].astype(i32)
    dst = edge_index[1].astype(i32)
    loops = jnp.arange(N, dtype=i32)
    pad = EP_PAD - EP
    srcs = jnp.concatenate([src, loops, jnp.zeros((pad,), i32)])
    dsts = jnp.concatenate([dst, loops, jnp.full((pad,), N, i32)])

    w1p = jnp.pad(W1, ((0, 0), (0, W1C - HID)))
    one1 = jnp.zeros((1, W1C), f32).at[0, HID].set(1.0)
    asv1 = jnp.pad(a_src1, (0, W1C - HID)).reshape(W1C, 1)
    adv1 = jnp.pad(a_dst1, (0, W1C - HID)).reshape(W1C, 1)
    w2p = jnp.pad(W2, ((0, 0), (0, W2C - HID2)))
    one2 = jnp.zeros((1, W2C), f32).at[0, HID2].set(1.0)
    asv2 = jnp.pad(a_src2, (0, W2C - HID2)).reshape(W2C, 1)
    adv2 = jnp.pad(a_dst2, (0, W2C - HID2)).reshape(W2C, 1)

    sc1 = _make_sc_layer(W1C, ((0, 1672), (1672, 1664), (3336, 1664)))
    sc2 = _make_sc_layer(
        W2C, ((0, 632), (632, 632), (1264, 632), (1896, 632), (2528, 632),
              (3160, 632), (3792, 632), (4424, 576)))

    ht1, as1, ad1 = _dense1(x, w1p, asv1, adv1, one1)
    zr1 = jnp.zeros((K, W1C), f32)
    acc1 = sc1(ht1, as1.reshape(N), ad1.reshape(N), srcs, dsts, zr1)

    ht2, as2, ad2 = _dense2(acc1, b1.reshape(1, HID), w2p, asv2, adv2, one2)
    zr2 = jnp.zeros((K, W2C), f32)
    acc2 = sc2(ht2, as2.reshape(N), ad2.reshape(N), srcs, dsts, zr2)

    return _final(acc2, b2.reshape(1, HID2), batch.reshape(N, 1).astype(i32),
                  fcW, fcb.reshape(1, OUT))
